# baseline (speedup 1.0000x reference)
import functools

import jax
import jax.numpy as jnp
from jax import lax
from jax.experimental import pallas as pl
from jax.experimental.pallas import tpu as pltpu
from jax.experimental.pallas import tpu_sc as plsc

NDIM = 32
CTX = 20
NW = 32
IW = 64
TBLK = 4096
OUTW = 128
KSLOTS = OUTW // NDIM
QROWS = TBLK // KSLOTS


def _tc_transpose(table_t):
    nrows = table_t.shape[1]
    grid = (nrows + TBLK - 1) // TBLK

    def body(in_ref, out_ref):
        rows = lax.broadcasted_iota(jnp.int32, (NDIM, NDIM), 0)
        cols = lax.broadcasted_iota(jnp.int32, (NDIM, NDIM), 1)
        eye = (rows == cols).astype(jnp.float32)
        slabs = []
        for k in range(KSLOTS):
            a_k = in_ref[:, pl.ds(k * QROWS, QROWS)]
            slabs.append(jax.lax.dot_general(
                a_k, eye, (((0,), (0,)), ((), ())),
                precision=jax.lax.Precision.HIGHEST,
                preferred_element_type=jnp.float32))
        out_ref[...] = jnp.concatenate(slabs, axis=1)

    return pl.pallas_call(
        body,
        grid=(grid,),
        in_specs=[pl.BlockSpec((NDIM, TBLK), lambda i: (0, i))],
        out_specs=pl.BlockSpec((QROWS, OUTW), lambda i: (i, 0)),
        out_shape=jax.ShapeDtypeStruct((grid * QROWS, OUTW), jnp.float32),
    )(table_t)


def _phys(r):
    return ((r >> 12) << 10) + (r & 1023)


def _coff(r):
    return ((r & 4095) >> 10) * NDIM


def _sc_kernel(batch):
    bpw = batch // NW
    cb = 16
    sc_chunks = bpw // cb
    rows = cb * CTX
    gi = rows // IW
    idx_rows = bpw * CTX // IW
    off_rows = sc_chunks * CTX // 8

    mesh = plsc.VectorSubcoreMesh(core_axis_name="c", subcore_axis_name="s")

    @functools.partial(
        pl.kernel,
        mesh=mesh,
        out_type=jax.ShapeDtypeStruct((batch,), jnp.float32),
        compiler_params=pltpu.CompilerParams(
            needs_layout_passes=False, use_tc_tiling_on_sc=False),
        scratch_types=[
            pltpu.VMEM((idx_rows, IW), jnp.int32),
            pltpu.VMEM((off_rows, 128), jnp.int32),
            pltpu.VMEM((sc_chunks, 16), jnp.int32),
            pltpu.VMEM((sc_chunks, 16), jnp.int32),
            pltpu.VMEM((rows, OUTW), jnp.float32),
            pltpu.VMEM((rows, OUTW), jnp.float32),
            pltpu.VMEM((cb, OUTW), jnp.float32),
            pltpu.VMEM((cb, OUTW), jnp.float32),
            pltpu.VMEM((bpw,), jnp.float32),
            pltpu.SemaphoreType.DMA,
            pltpu.SemaphoreType.DMA,
        ],
    )
    def body(xq2d, off2d, tq2d, toff2d, embed4, embed_out4, out,
             xq_v, off_v, tq_v, toff_v, buf_a, buf_b, tbuf_a, tbuf_b,
             out_v, sem_a, sem_b):
        wid = lax.axis_index("s") * 2 + lax.axis_index("c")
        base = wid * bpw

        pltpu.sync_copy(xq2d.at[pl.ds(wid * idx_rows, idx_rows)], xq_v)
        pltpu.sync_copy(off2d.at[pl.ds(wid * off_rows, off_rows)], off_v)
        pltpu.sync_copy(tq2d.at[pl.ds(wid * sc_chunks, sc_chunks)], tq_v)
        pltpu.sync_copy(toff2d.at[pl.ds(wid * sc_chunks, sc_chunks)], toff_v)

        bufs = (buf_a, buf_b)
        tbufs = (tbuf_a, tbuf_b)
        sems = (sem_a, sem_b)

        def fire(s):
            dmas = []
            buf = bufs[s % 2]
            sem = sems[s % 2]
            for g in range(gi):
                dmas.append(pltpu.async_copy(
                    embed4.at[xq_v.at[s * gi + g]],
                    buf.at[pl.ds(g * IW, IW)], sem))
            dmas.append(pltpu.async_copy(
                embed_out4.at[tq_v.at[s]], tbufs[s % 2], sem))
            return dmas

        inflight = fire(0)
        lanes = lax.iota(jnp.int32, 16)

        for s in range(sc_chunks):
            nxt = fire(s + 1) if s + 1 < sc_chunks else []
            for d in inflight:
                d.wait()
            inflight = nxt
            buf = bufs[s % 2]
            tbuf = tbufs[s % 2]

            offs = [
                off_v[(s * CTX + j) // 8, pl.ds(((s * CTX + j) % 8) * 16, 16)]
                for j in range(CTX)]
            toffv = toff_v[s, pl.ds(0, 16)]
            rowv = lanes * CTX

            def dot_dim(d, acc):
                pool = plsc.load_gather(buf, [rowv, offs[0] + d])
                for j in range(1, CTX):
                    pool = pool + plsc.load_gather(
                        buf, [rowv + j, offs[j] + d])
                tgt = plsc.load_gather(tbuf, [lanes, toffv + d])
                return acc + pool * tgt
            acc = lax.fori_loop(0, NDIM, dot_dim, lanes * jnp.float32(0),
                                unroll=False)
            out_v[pl.ds(s * cb, cb)] = acc

        pltpu.sync_copy(out_v, out.at[pl.ds(base, bpw)])

    return body


def kernel(x, target_id, embed, embed_out):
    batch, ctx = x.shape
    assert ctx == CTX
    xq2d = _phys(x).reshape(batch * CTX // IW, IW)
    off = _coff(x).reshape(batch // 16, 16, CTX)
    off2d = off.transpose(0, 2, 1).reshape(batch // 16 * CTX // 8, 128)
    tq2d = _phys(target_id).reshape(batch // 16, 16)
    toff2d = _coff(target_id).reshape(batch // 16, 16)
    embed4 = _tc_transpose(embed.T)
    embed_out4 = _tc_transpose(embed_out.T)
    return _sc_kernel(batch)(xq2d, off2d, tq2d, toff2d, embed4, embed_out4)


# speedup vs baseline: 3.1330x; 3.1330x over previous
import functools

import jax
import jax.numpy as jnp
from jax import lax
from jax.experimental import pallas as pl
from jax.experimental.pallas import tpu as pltpu
from jax.experimental.pallas import tpu_sc as plsc

NDIM = 32
CTX = 20
NW = 32
IW = 64
TBLK = 16384
OUTW = 128
KSLOTS = OUTW // NDIM
QROWS = TBLK // KSLOTS


def _tc_transpose(table_t):
    nrows = table_t.shape[1]
    grid = (nrows + TBLK - 1) // TBLK

    def body(in_ref, out_ref):
        for t in range(QROWS // 128):
            s = jnp.concatenate(
                [in_ref[:, pl.ds(k * QROWS + t * 128, 128)]
                 for k in range(KSLOTS)], axis=0)
            out_ref[pl.ds(t * 128, 128), :] = jnp.swapaxes(s, 0, 1)

    return pl.pallas_call(
        body,
        grid=(grid,),
        in_specs=[pl.BlockSpec((NDIM, TBLK), lambda i: (0, i))],
        out_specs=pl.BlockSpec((QROWS, OUTW), lambda i: (i, 0)),
        out_shape=jax.ShapeDtypeStruct((grid * QROWS, OUTW), jnp.float32),
    )(table_t)


def _phys(r):
    return (r // TBLK) * QROWS + (r % QROWS)


def _coff(r):
    return ((r % TBLK) // QROWS) * NDIM


def _sc_kernel(batch):
    bpw = batch // NW
    cb = 16
    sc_chunks = bpw // cb
    rows = cb * CTX
    gi = rows // IW
    idx_rows = bpw * CTX // IW
    off_rows = sc_chunks * CTX // 8

    mesh = plsc.VectorSubcoreMesh(core_axis_name="c", subcore_axis_name="s")

    @functools.partial(
        pl.kernel,
        mesh=mesh,
        out_type=jax.ShapeDtypeStruct((batch,), jnp.float32),
        compiler_params=pltpu.CompilerParams(
            needs_layout_passes=False, use_tc_tiling_on_sc=False),
        scratch_types=[
            pltpu.VMEM((idx_rows, IW), jnp.int32),
            pltpu.VMEM((off_rows, 128), jnp.int32),
            pltpu.VMEM((sc_chunks, 16), jnp.int32),
            pltpu.VMEM((sc_chunks, 16), jnp.int32),
            pltpu.VMEM((rows, OUTW), jnp.float32),
            pltpu.VMEM((rows, OUTW), jnp.float32),
            pltpu.VMEM((cb, OUTW), jnp.float32),
            pltpu.VMEM((cb, OUTW), jnp.float32),
            pltpu.VMEM((bpw,), jnp.float32),
            pltpu.SemaphoreType.DMA,
            pltpu.SemaphoreType.DMA,
        ],
    )
    def body(xq2d, off2d, tq2d, toff2d, embed4, embed_out4, out,
             xq_v, off_v, tq_v, toff_v, buf_a, buf_b, tbuf_a, tbuf_b,
             out_v, sem_a, sem_b):
        wid = lax.axis_index("s") * 2 + lax.axis_index("c")
        base = wid * bpw

        pltpu.sync_copy(xq2d.at[pl.ds(wid * idx_rows, idx_rows)], xq_v)
        pltpu.sync_copy(off2d.at[pl.ds(wid * off_rows, off_rows)], off_v)
        pltpu.sync_copy(tq2d.at[pl.ds(wid * sc_chunks, sc_chunks)], tq_v)
        pltpu.sync_copy(toff2d.at[pl.ds(wid * sc_chunks, sc_chunks)], toff_v)

        bufs = (buf_a, buf_b)
        tbufs = (tbuf_a, tbuf_b)
        sems = (sem_a, sem_b)

        def fire(s):
            dmas = []
            buf = bufs[s % 2]
            sem = sems[s % 2]
            for g in range(gi):
                dmas.append(pltpu.async_copy(
                    embed4.at[xq_v.at[s * gi + g]],
                    buf.at[pl.ds(g * IW, IW)], sem))
            dmas.append(pltpu.async_copy(
                embed_out4.at[tq_v.at[s]], tbufs[s % 2], sem))
            return dmas

        inflight = fire(0)
        lanes = lax.iota(jnp.int32, 16)

        for s in range(sc_chunks):
            nxt = fire(s + 1) if s + 1 < sc_chunks else []
            for d in inflight:
                d.wait()
            inflight = nxt
            buf = bufs[s % 2]
            tbuf = tbufs[s % 2]

            offs = [
                off_v[(s * CTX + j) // 8, pl.ds(((s * CTX + j) % 8) * 16, 16)]
                for j in range(CTX)]
            toffv = toff_v[s, pl.ds(0, 16)]
            rowv = lanes * CTX

            def dot_dim(d, acc):
                pool = plsc.load_gather(buf, [rowv, offs[0] + d])
                for j in range(1, CTX):
                    pool = pool + plsc.load_gather(
                        buf, [rowv + j, offs[j] + d])
                tgt = plsc.load_gather(tbuf, [lanes, toffv + d])
                return acc + pool * tgt
            acc = lax.fori_loop(0, NDIM, dot_dim, lanes * jnp.float32(0),
                                unroll=False)
            out_v[pl.ds(s * cb, cb)] = acc

        pltpu.sync_copy(out_v, out.at[pl.ds(base, bpw)])

    return body


def kernel(x, target_id, embed, embed_out):
    batch, ctx = x.shape
    assert ctx == CTX
    xq2d = _phys(x).reshape(batch * CTX // IW, IW)
    off = _coff(x).reshape(batch // 16, 16, CTX)
    off2d = off.transpose(0, 2, 1).reshape(batch // 16 * CTX // 8, 128)
    tq2d = _phys(target_id).reshape(batch // 16, 16)
    toff2d = _coff(target_id).reshape(batch // 16, 16)
    embed4 = _tc_transpose(embed.T)
    embed_out4 = _tc_transpose(embed_out.T)
    return _sc_kernel(batch)(xq2d, off2d, tq2d, toff2d, embed4, embed_out4)


# speedup vs baseline: 3.4434x; 1.0991x over previous
import functools

import jax
import jax.numpy as jnp
from jax import lax
from jax.experimental import pallas as pl
from jax.experimental.pallas import tpu as pltpu
from jax.experimental.pallas import tpu_sc as plsc

NDIM = 32
CTX = 20
NW = 32
IW = 64
TBLK = 16384
OUTW = 128
KSLOTS = OUTW // NDIM
QROWS = TBLK // KSLOTS


def _tc_transpose(table_t):
    nrows = table_t.shape[1]
    grid = (nrows + TBLK - 1) // TBLK

    def body(in_ref, out_ref):
        for t in range(QROWS // 128):
            s = jnp.concatenate(
                [in_ref[:, pl.ds(k * QROWS + t * 128, 128)]
                 for k in range(KSLOTS)], axis=0)
            out_ref[pl.ds(t * 128, 128), :] = jnp.swapaxes(s, 0, 1)

    return pl.pallas_call(
        body,
        grid=(grid,),
        in_specs=[pl.BlockSpec((NDIM, TBLK), lambda i: (0, i))],
        out_specs=pl.BlockSpec((QROWS, OUTW), lambda i: (i, 0)),
        out_shape=jax.ShapeDtypeStruct((grid * QROWS, OUTW), jnp.float32),
    )(table_t)


def _phys(r):
    return (r // TBLK) * QROWS + (r % QROWS)


def _coff(r):
    return ((r % TBLK) // QROWS) * NDIM


def _sc_kernel(batch):
    bpw = batch // NW
    cb = 16
    sc_chunks = bpw // cb
    rows = cb * CTX
    gi = rows // IW
    idx_rows = bpw * CTX // IW
    off_rows = sc_chunks * CTX // 8

    mesh = plsc.VectorSubcoreMesh(core_axis_name="c", subcore_axis_name="s")

    def _vphys(v):
        return lax.shift_right_logical(v, 14) * QROWS + (v & (QROWS - 1))

    def _voff(v):
        return ((lax.shift_right_logical(v, 12) & (KSLOTS - 1))) * NDIM

    @functools.partial(
        pl.kernel,
        mesh=mesh,
        out_type=jax.ShapeDtypeStruct((batch,), jnp.float32),
        compiler_params=pltpu.CompilerParams(
            needs_layout_passes=False, use_tc_tiling_on_sc=False),
        scratch_types=[
            pltpu.VMEM((CTX, bpw), jnp.int32),
            pltpu.VMEM((idx_rows, IW), jnp.int32),
            pltpu.VMEM((bpw,), jnp.int32),
            pltpu.VMEM((sc_chunks, 16), jnp.int32),
            pltpu.VMEM((rows, OUTW), jnp.float32),
            pltpu.VMEM((rows, OUTW), jnp.float32),
            pltpu.VMEM((cb, OUTW), jnp.float32),
            pltpu.VMEM((cb, OUTW), jnp.float32),
            pltpu.VMEM((bpw,), jnp.float32),
            pltpu.SemaphoreType.DMA,
            pltpu.SemaphoreType.DMA,
        ],
    )
    def body(x_t, tid, embed4, embed_out4, out,
             x_v, xq_v, t_v, tq_v, buf_a, buf_b, tbuf_a, tbuf_b,
             out_v, sem_a, sem_b):
        wid = lax.axis_index("s") * 2 + lax.axis_index("c")
        base = wid * bpw

        pltpu.sync_copy(x_t.at[:, pl.ds(base, bpw)], x_v)
        pltpu.sync_copy(tid.at[pl.ds(base, bpw)], t_v)

        def prep(s, _):
            tq_v[s, pl.ds(0, 16)] = _vphys(t_v[pl.ds(s * 16, 16)])
            for j in range(CTX):
                xq_v[s * gi + j // 4, pl.ds((j % 4) * 16, 16)] = _vphys(
                    x_v[j, pl.ds(s * 16, 16)])
            return 0
        lax.fori_loop(0, sc_chunks, prep, 0, unroll=False)

        bufs = (buf_a, buf_b)
        tbufs = (tbuf_a, tbuf_b)
        sems = (sem_a, sem_b)

        def fire(s):
            dmas = []
            buf = bufs[s % 2]
            sem = sems[s % 2]
            for g in range(gi):
                dmas.append(pltpu.async_copy(
                    embed4.at[xq_v.at[s * gi + g]],
                    buf.at[pl.ds(g * IW, IW)], sem))
            dmas.append(pltpu.async_copy(
                embed_out4.at[tq_v.at[s]], tbufs[s % 2], sem))
            return dmas

        inflight = fire(0)
        lanes = lax.iota(jnp.int32, 16)

        for s in range(sc_chunks):
            nxt = fire(s + 1) if s + 1 < sc_chunks else []
            for d in inflight:
                d.wait()
            inflight = nxt
            buf = bufs[s % 2]
            tbuf = tbufs[s % 2]

            offs = [_voff(x_v[j, pl.ds(s * 16, 16)]) for j in range(CTX)]
            toffv = _voff(t_v[pl.ds(s * 16, 16)])

            def dot_dim(d, acc):
                pool = plsc.load_gather(buf, [lanes, offs[0] + d])
                for j in range(1, CTX):
                    pool = pool + plsc.load_gather(
                        buf, [lanes + j * 16, offs[j] + d])
                tgt = plsc.load_gather(tbuf, [lanes, toffv + d])
                return acc + pool * tgt
            acc = lax.fori_loop(0, NDIM, dot_dim, lanes * jnp.float32(0),
                                unroll=False)
            out_v[pl.ds(s * cb, cb)] = acc

        pltpu.sync_copy(out_v, out.at[pl.ds(base, bpw)])

    return body


def kernel(x, target_id, embed, embed_out):
    batch, ctx = x.shape
    assert ctx == CTX
    x_t = x.T
    embed4 = _tc_transpose(embed.T)
    embed_out4 = _tc_transpose(embed_out.T)
    return _sc_kernel(batch)(x_t, target_id, embed4, embed_out4)


# speedup vs baseline: 3.5775x; 1.0390x over previous
import functools

import jax
import jax.numpy as jnp
from jax import lax
from jax.experimental import pallas as pl
from jax.experimental.pallas import tpu as pltpu
from jax.experimental.pallas import tpu_sc as plsc

NDIM = 32
CTX = 20
NW = 32
IW = 64
TBLK = 16384
OUTW = 128
KSLOTS = OUTW // NDIM
QROWS = TBLK // KSLOTS

_MESH = dict(core_axis_name="c", subcore_axis_name="s")


def _tc_transpose(table_t):
    nrows = table_t.shape[1]
    grid = (nrows + TBLK - 1) // TBLK

    def body(in_ref, out_ref):
        for t in range(QROWS // 128):
            s = jnp.concatenate(
                [in_ref[:, pl.ds(k * QROWS + t * 128, 128)]
                 for k in range(KSLOTS)], axis=0)
            out_ref[pl.ds(t * 128, 128), :] = jnp.swapaxes(s, 0, 1)

    return pl.pallas_call(
        body,
        grid=(grid,),
        in_specs=[pl.BlockSpec((NDIM, TBLK), lambda i: (0, i))],
        out_specs=pl.BlockSpec((QROWS, OUTW), lambda i: (i, 0)),
        out_shape=jax.ShapeDtypeStruct((grid * QROWS, OUTW), jnp.float32),
    )(table_t)


def _vphys(v):
    return lax.shift_right_logical(v, 14) * QROWS + (v & (QROWS - 1))


def _voff(v):
    return (lax.shift_right_logical(v, 12) & (KSLOTS - 1)) * NDIM


def _pool_kernel(batch):
    bpw = batch // NW
    cb = 16
    sc_chunks = bpw // cb
    rows = cb * CTX
    gi = rows // IW
    idx_rows = bpw * CTX // IW

    mesh = plsc.VectorSubcoreMesh(**_MESH)

    @functools.partial(
        pl.kernel,
        mesh=mesh,
        out_type=jax.ShapeDtypeStruct((batch // KSLOTS, OUTW), jnp.float32),
        compiler_params=pltpu.CompilerParams(
            needs_layout_passes=False, use_tc_tiling_on_sc=False),
        scratch_types=[
            pltpu.VMEM((CTX, bpw), jnp.int32),
            pltpu.VMEM((idx_rows, IW), jnp.int32),
            pltpu.VMEM((rows, OUTW), jnp.float32),
            pltpu.VMEM((rows, OUTW), jnp.float32),
            pltpu.VMEM((bpw // KSLOTS, OUTW), jnp.float32),
            pltpu.SemaphoreType.DMA,
            pltpu.SemaphoreType.DMA,
        ],
    )
    def body(x_t, embed4, out, x_v, xq_v, buf_a, buf_b, pool_v,
             sem_a, sem_b):
        wid = lax.axis_index("s") * 2 + lax.axis_index("c")
        base = wid * bpw

        pltpu.sync_copy(x_t.at[:, pl.ds(base, bpw)], x_v)

        def prep(s, _):
            for j in range(CTX):
                xq_v[s * gi + j // 4, pl.ds((j % 4) * 16, 16)] = _vphys(
                    x_v[j, pl.ds(s * 16, 16)])
            return 0
        lax.fori_loop(0, sc_chunks, prep, 0, unroll=False)

        bufs = (buf_a, buf_b)
        sems = (sem_a, sem_b)

        def fire(s):
            buf = bufs[s % 2]
            sem = sems[s % 2]
            return [pltpu.async_copy(
                embed4.at[xq_v.at[s * gi + g]],
                buf.at[pl.ds(g * IW, IW)], sem) for g in range(gi)]

        inflight = fire(0)
        lanes = lax.iota(jnp.int32, 16)
        prow = lax.shift_right_logical(lanes, 2)
        pcol = (lanes & 3) * NDIM

        for s in range(sc_chunks):
            nxt = fire(s + 1) if s + 1 < sc_chunks else []
            for d in inflight:
                d.wait()
            inflight = nxt
            buf = bufs[s % 2]

            offs = [_voff(x_v[j, pl.ds(s * 16, 16)]) for j in range(CTX)]

            def pool_dim(d, _):
                acc = plsc.load_gather(buf, [lanes, offs[0] + d])
                for j in range(1, CTX):
                    acc = acc + plsc.load_gather(
                        buf, [lanes + j * 16, offs[j] + d])
                plsc.store_scatter(pool_v, [s * 4 + prow, pcol + d], acc)
                return 0
            lax.fori_loop(0, NDIM, pool_dim, 0, unroll=False)

        pltpu.sync_copy(pool_v, out.at[pl.ds(wid * (bpw // KSLOTS),
                                             bpw // KSLOTS)])

    return body


def _dot_kernel(batch):
    bpw = batch // NW
    cb = 16
    sc_chunks = bpw // cb
    prows = bpw // KSLOTS

    mesh = plsc.VectorSubcoreMesh(**_MESH)

    @functools.partial(
        pl.kernel,
        mesh=mesh,
        out_type=jax.ShapeDtypeStruct((batch,), jnp.float32),
        compiler_params=pltpu.CompilerParams(
            needs_layout_passes=False, use_tc_tiling_on_sc=False),
        scratch_types=[
            pltpu.VMEM((bpw,), jnp.int32),
            pltpu.VMEM((sc_chunks, 16), jnp.int32),
            pltpu.VMEM((prows, OUTW), jnp.float32),
            pltpu.VMEM((cb, OUTW), jnp.float32),
            pltpu.VMEM((cb, OUTW), jnp.float32),
            pltpu.VMEM((bpw,), jnp.float32),
            pltpu.SemaphoreType.DMA,
            pltpu.SemaphoreType.DMA,
        ],
    )
    def body(tid, pooled4, embed_out4, out, t_v, tq_v, pool_v,
             tbuf_a, tbuf_b, out_v, sem_a, sem_b):
        wid = lax.axis_index("s") * 2 + lax.axis_index("c")
        base = wid * bpw

        pltpu.sync_copy(tid.at[pl.ds(base, bpw)], t_v)
        pltpu.sync_copy(pooled4.at[pl.ds(wid * prows, prows)], pool_v)

        def prep(s, _):
            tq_v[s, pl.ds(0, 16)] = _vphys(t_v[pl.ds(s * 16, 16)])
            return 0
        lax.fori_loop(0, sc_chunks, prep, 0, unroll=False)

        tbufs = (tbuf_a, tbuf_b)
        sems = (sem_a, sem_b)

        def fire(s):
            return [pltpu.async_copy(
                embed_out4.at[tq_v.at[s]], tbufs[s % 2], sems[s % 2])]

        inflight = fire(0)
        lanes = lax.iota(jnp.int32, 16)
        prow = lax.shift_right_logical(lanes, 2)
        pcol = (lanes & 3) * NDIM

        for s in range(sc_chunks):
            nxt = fire(s + 1) if s + 1 < sc_chunks else []
            for d in inflight:
                d.wait()
            inflight = nxt
            tbuf = tbufs[s % 2]

            toffv = _voff(t_v[pl.ds(s * 16, 16)])

            def dot_dim(d, acc):
                pooled = plsc.load_gather(pool_v, [s * 4 + prow, pcol + d])
                tgt = plsc.load_gather(tbuf, [lanes, toffv + d])
                return acc + pooled * tgt
            acc = lax.fori_loop(0, NDIM, dot_dim, lanes * jnp.float32(0),
                                unroll=False)
            out_v[pl.ds(s * cb, cb)] = acc

        pltpu.sync_copy(out_v, out.at[pl.ds(base, bpw)])

    return body


def kernel(x, target_id, embed, embed_out):
    batch, ctx = x.shape
    assert ctx == CTX
    x_t = x.T
    embed4 = _tc_transpose(embed.T)
    pooled4 = _pool_kernel(batch)(x_t, embed4)
    embed_out4 = _tc_transpose(embed_out.T)
    return _dot_kernel(batch)(target_id, pooled4, embed_out4)


# speedup vs baseline: 3.5879x; 1.0029x over previous
import functools

import jax
import jax.numpy as jnp
from jax import lax
from jax.experimental import pallas as pl
from jax.experimental.pallas import tpu as pltpu
from jax.experimental.pallas import tpu_sc as plsc

NDIM = 32
CTX = 20
NW = 32
IW = 64
TBLK = 16384
OUTW = 128
KSLOTS = OUTW // NDIM
QROWS = TBLK // KSLOTS

_MESH = dict(core_axis_name="c", subcore_axis_name="s")


def _tc_transpose(table_t):
    nrows = table_t.shape[1]
    grid = (nrows + TBLK - 1) // TBLK

    def body(in_ref, out_ref):
        for t in range(QROWS // 128):
            s = jnp.concatenate(
                [in_ref[:, pl.ds(k * QROWS + t * 128, 128)]
                 for k in range(KSLOTS)], axis=0)
            out_ref[pl.ds(t * 128, 128), :] = jnp.swapaxes(s, 0, 1)

    return pl.pallas_call(
        body,
        grid=(grid,),
        in_specs=[pl.BlockSpec((NDIM, TBLK), lambda i: (0, i))],
        out_specs=pl.BlockSpec((QROWS, OUTW), lambda i: (i, 0)),
        out_shape=jax.ShapeDtypeStruct((grid * QROWS, OUTW), jnp.float32),
    )(table_t)


def _vphys(v):
    return lax.shift_right_logical(v, 14) * QROWS + (v & (QROWS - 1))


def _voff(v):
    return (lax.shift_right_logical(v, 12) & (KSLOTS - 1)) * NDIM


def _pool_kernel(batch):
    bpw = batch // NW
    cb = 16
    sc_chunks = bpw // cb
    rows = cb * CTX
    gi = rows // IW
    idx_rows = bpw * CTX // IW

    mesh = plsc.VectorSubcoreMesh(**_MESH)

    @functools.partial(
        pl.kernel,
        mesh=mesh,
        out_type=jax.ShapeDtypeStruct((batch // KSLOTS, OUTW), jnp.float32),
        compiler_params=pltpu.CompilerParams(
            needs_layout_passes=False, use_tc_tiling_on_sc=False),
        scratch_types=[
            pltpu.VMEM((CTX, bpw), jnp.int32),
            pltpu.VMEM((sc_chunks, rows), jnp.int32),
            pltpu.VMEM((rows, OUTW), jnp.float32),
            pltpu.VMEM((rows, OUTW), jnp.float32),
            pltpu.VMEM((bpw // KSLOTS, OUTW), jnp.float32),
            pltpu.SemaphoreType.DMA,
            pltpu.SemaphoreType.DMA,
        ],
    )
    def body(x_t, embed4, out, x_v, xq_v, buf_a, buf_b, pool_v,
             sem_a, sem_b):
        wid = lax.axis_index("s") * 2 + lax.axis_index("c")
        base = wid * bpw

        pltpu.sync_copy(x_t.at[:, pl.ds(base, bpw)], x_v)

        def prep(s, _):
            for j in range(CTX):
                xq_v[s, pl.ds(j * 16, 16)] = _vphys(
                    x_v[j, pl.ds(s * 16, 16)])
            return 0
        lax.fori_loop(0, sc_chunks, prep, 0, unroll=False)

        bufs = (buf_a, buf_b)
        sems = (sem_a, sem_b)

        def fire(s):
            return [pltpu.async_copy(
                embed4.at[xq_v.at[s]], bufs[s % 2], sems[s % 2])]

        inflight = fire(0)
        lanes = lax.iota(jnp.int32, 16)
        prow = lax.shift_right_logical(lanes, 2)
        pcol = (lanes & 3) * NDIM

        for s in range(sc_chunks):
            nxt = fire(s + 1) if s + 1 < sc_chunks else []
            for d in inflight:
                d.wait()
            inflight = nxt
            buf = bufs[s % 2]

            offs = [_voff(x_v[j, pl.ds(s * 16, 16)]) for j in range(CTX)]

            def pool_dim(d, _):
                acc = plsc.load_gather(buf, [lanes, offs[0] + d])
                for j in range(1, CTX):
                    acc = acc + plsc.load_gather(
                        buf, [lanes + j * 16, offs[j] + d])
                plsc.store_scatter(pool_v, [s * 4 + prow, pcol + d], acc)
                return 0
            lax.fori_loop(0, NDIM, pool_dim, 0, unroll=False)

        pltpu.sync_copy(pool_v, out.at[pl.ds(wid * (bpw // KSLOTS),
                                             bpw // KSLOTS)])

    return body


def _dot_kernel(batch):
    bpw = batch // NW
    cb = 16
    sc_chunks = bpw // cb
    prows = bpw // KSLOTS

    mesh = plsc.VectorSubcoreMesh(**_MESH)

    @functools.partial(
        pl.kernel,
        mesh=mesh,
        out_type=jax.ShapeDtypeStruct((batch,), jnp.float32),
        compiler_params=pltpu.CompilerParams(
            needs_layout_passes=False, use_tc_tiling_on_sc=False),
        scratch_types=[
            pltpu.VMEM((bpw,), jnp.int32),
            pltpu.VMEM((sc_chunks, 16), jnp.int32),
            pltpu.VMEM((prows, OUTW), jnp.float32),
            pltpu.VMEM((cb, OUTW), jnp.float32),
            pltpu.VMEM((cb, OUTW), jnp.float32),
            pltpu.VMEM((bpw,), jnp.float32),
            pltpu.SemaphoreType.DMA,
            pltpu.SemaphoreType.DMA,
        ],
    )
    def body(tid, pooled4, embed_out4, out, t_v, tq_v, pool_v,
             tbuf_a, tbuf_b, out_v, sem_a, sem_b):
        wid = lax.axis_index("s") * 2 + lax.axis_index("c")
        base = wid * bpw

        pltpu.sync_copy(tid.at[pl.ds(base, bpw)], t_v)
        pltpu.sync_copy(pooled4.at[pl.ds(wid * prows, prows)], pool_v)

        def prep(s, _):
            tq_v[s, pl.ds(0, 16)] = _vphys(t_v[pl.ds(s * 16, 16)])
            return 0
        lax.fori_loop(0, sc_chunks, prep, 0, unroll=False)

        tbufs = (tbuf_a, tbuf_b)
        sems = (sem_a, sem_b)

        def fire(s):
            return [pltpu.async_copy(
                embed_out4.at[tq_v.at[s]], tbufs[s % 2], sems[s % 2])]

        inflight = fire(0)
        lanes = lax.iota(jnp.int32, 16)
        prow = lax.shift_right_logical(lanes, 2)
        pcol = (lanes & 3) * NDIM

        for s in range(sc_chunks):
            nxt = fire(s + 1) if s + 1 < sc_chunks else []
            for d in inflight:
                d.wait()
            inflight = nxt
            tbuf = tbufs[s % 2]

            toffv = _voff(t_v[pl.ds(s * 16, 16)])

            def dot_dim(d, acc):
                pooled = plsc.load_gather(pool_v, [s * 4 + prow, pcol + d])
                tgt = plsc.load_gather(tbuf, [lanes, toffv + d])
                return acc + pooled * tgt
            acc = lax.fori_loop(0, NDIM, dot_dim, lanes * jnp.float32(0),
                                unroll=False)
            out_v[pl.ds(s * cb, cb)] = acc

        pltpu.sync_copy(out_v, out.at[pl.ds(base, bpw)])

    return body


def kernel(x, target_id, embed, embed_out):
    batch, ctx = x.shape
    assert ctx == CTX
    x_t = x.T
    embed4 = _tc_transpose(embed.T)
    pooled4 = _pool_kernel(batch)(x_t, embed4)
    embed_out4 = _tc_transpose(embed_out.T)
    return _dot_kernel(batch)(target_id, pooled4, embed_out4)


# speedup vs baseline: 3.5990x; 1.0031x over previous
import functools

import jax
import jax.numpy as jnp
from jax import lax
from jax.experimental import pallas as pl
from jax.experimental.pallas import tpu as pltpu
from jax.experimental.pallas import tpu_sc as plsc

NDIM = 32
CTX = 20
NW = 32
IW = 64
TBLK = 16384
OUTW = 128
KSLOTS = OUTW // NDIM
QROWS = TBLK // KSLOTS

_MESH = dict(core_axis_name="c", subcore_axis_name="s")


def _tc_transpose(table_t):
    nrows = table_t.shape[1]
    grid = (nrows + TBLK - 1) // TBLK

    def body(in_ref, out_ref):
        for t in range(QROWS // 128):
            s = jnp.concatenate(
                [in_ref[:, pl.ds(k * QROWS + t * 128, 128)]
                 for k in range(KSLOTS)], axis=0)
            out_ref[pl.ds(t * 128, 128), :] = jnp.swapaxes(s, 0, 1)

    return pl.pallas_call(
        body,
        grid=(grid,),
        in_specs=[pl.BlockSpec((NDIM, TBLK), lambda i: (0, i))],
        out_specs=pl.BlockSpec((QROWS, OUTW), lambda i: (i, 0)),
        out_shape=jax.ShapeDtypeStruct((grid * QROWS, OUTW), jnp.float32),
    )(table_t)


def _vphys(v):
    return lax.shift_right_logical(v, 14) * QROWS + (v & (QROWS - 1))


def _voff(v):
    return (lax.shift_right_logical(v, 12) & (KSLOTS - 1)) * NDIM


def _pool_kernel(batch):
    bpw = batch // NW
    cb = 16
    sc_chunks = bpw // cb
    rows = cb * CTX
    gi = rows // IW
    idx_rows = bpw * CTX // IW

    mesh = plsc.VectorSubcoreMesh(**_MESH)

    @functools.partial(
        pl.kernel,
        mesh=mesh,
        out_type=jax.ShapeDtypeStruct((batch // KSLOTS, OUTW), jnp.float32),
        compiler_params=pltpu.CompilerParams(
            needs_layout_passes=False, use_tc_tiling_on_sc=False),
        scratch_types=[
            pltpu.VMEM((CTX, bpw), jnp.int32),
            pltpu.VMEM((sc_chunks, rows), jnp.int32),
            pltpu.VMEM((rows, OUTW), jnp.float32),
            pltpu.VMEM((rows, OUTW), jnp.float32),
            pltpu.VMEM((bpw // KSLOTS, OUTW), jnp.float32),
            pltpu.SemaphoreType.DMA,
            pltpu.SemaphoreType.DMA,
        ],
    )
    def body(x_t, embed4, out, x_v, xq_v, buf_a, buf_b, pool_v,
             sem_a, sem_b):
        wid = lax.axis_index("s") * 2 + lax.axis_index("c")
        base = wid * bpw

        pltpu.sync_copy(x_t.at[:, pl.ds(base, bpw)], x_v)

        def prep(s, _):
            for j in range(CTX):
                xq_v[s, pl.ds(j * 16, 16)] = _vphys(
                    x_v[j, pl.ds(s * 16, 16)])
            return 0
        lax.fori_loop(0, sc_chunks, prep, 0, unroll=False)

        bufs = (buf_a, buf_b)
        sems = (sem_a, sem_b)

        def fire(s):
            return [pltpu.async_copy(
                embed4.at[xq_v.at[s]], bufs[s % 2], sems[s % 2])]

        inflight = fire(0)
        lanes = lax.iota(jnp.int32, 16)
        prow = lax.shift_right_logical(lanes, 2)
        pcol = (lanes & 3) * NDIM

        for s in range(sc_chunks):
            nxt = fire(s + 1) if s + 1 < sc_chunks else []
            for d in inflight:
                d.wait()
            inflight = nxt
            buf = bufs[s % 2]

            offs = [_voff(x_v[j, pl.ds(s * 16, 16)]) for j in range(CTX)]

            def pool_dim(d, _):
                acc = plsc.load_gather(buf, [lanes, offs[0] + d])
                for j in range(1, CTX):
                    acc = acc + plsc.load_gather(
                        buf, [lanes + j * 16, offs[j] + d])
                plsc.store_scatter(pool_v, [s * 4 + prow, pcol + d], acc)
                return 0
            lax.fori_loop(0, NDIM, pool_dim, 0, unroll=False)

        pltpu.sync_copy(pool_v, out.at[pl.ds(wid * (bpw // KSLOTS),
                                             bpw // KSLOTS)])

    return body


def _dot_kernel(batch):
    bpw = batch // NW
    cb = 16
    sc_chunks = bpw // cb
    prows = bpw // KSLOTS

    mesh = plsc.VectorSubcoreMesh(**_MESH)

    @functools.partial(
        pl.kernel,
        mesh=mesh,
        out_type=jax.ShapeDtypeStruct((batch,), jnp.float32),
        compiler_params=pltpu.CompilerParams(
            needs_layout_passes=False, use_tc_tiling_on_sc=False),
        scratch_types=[
            pltpu.VMEM((bpw,), jnp.int32),
            pltpu.VMEM((sc_chunks, 16), jnp.int32),
            pltpu.VMEM((prows, OUTW), jnp.float32),
            pltpu.VMEM((bpw, OUTW), jnp.float32),
            pltpu.VMEM((bpw,), jnp.float32),
            pltpu.SemaphoreType.DMA,
        ],
    )
    def body(tid, pooled4, embed_out4, out, t_v, tq_v, pool_v,
             tbuf, out_v, sem):
        wid = lax.axis_index("s") * 2 + lax.axis_index("c")
        base = wid * bpw

        pltpu.sync_copy(tid.at[pl.ds(base, bpw)], t_v)

        def prep(s, _):
            tq_v[s, pl.ds(0, 16)] = _vphys(t_v[pl.ds(s * 16, 16)])
            return 0
        lax.fori_loop(0, sc_chunks, prep, 0, unroll=False)

        dmas = [pltpu.async_copy(
            embed_out4.at[tq_v.at[s]], tbuf.at[pl.ds(s * cb, cb)], sem)
            for s in range(sc_chunks)]
        pltpu.sync_copy(pooled4.at[pl.ds(wid * prows, prows)], pool_v)
        for d in dmas:
            d.wait()

        lanes = lax.iota(jnp.int32, 16)
        prow = lax.shift_right_logical(lanes, 2)
        pcol = (lanes & 3) * NDIM

        for s in range(sc_chunks):
            toffv = _voff(t_v[pl.ds(s * 16, 16)])

            def dot_dim(d, acc):
                pooled = plsc.load_gather(pool_v, [s * 4 + prow, pcol + d])
                tgt = plsc.load_gather(
                    tbuf, [s * cb + lanes, toffv + d])
                return acc + pooled * tgt
            acc = lax.fori_loop(0, NDIM, dot_dim, lanes * jnp.float32(0),
                                unroll=False)
            out_v[pl.ds(s * cb, cb)] = acc

        pltpu.sync_copy(out_v, out.at[pl.ds(base, bpw)])

    return body


def kernel(x, target_id, embed, embed_out):
    batch, ctx = x.shape
    assert ctx == CTX
    x_t = x.T
    embed4 = _tc_transpose(embed.T)
    pooled4 = _pool_kernel(batch)(x_t, embed4)
    embed_out4 = _tc_transpose(embed_out.T)
    return _dot_kernel(batch)(target_id, pooled4, embed_out4)
